# baseline (speedup 1.0000x reference)
import functools

import jax
import jax.numpy as jnp
from jax import lax
from jax.experimental import pallas as pl
from jax.experimental.pallas import tpu as pltpu
from jax.experimental.pallas import tpu_sc as plsc

B, V, D, H = 16384, 100000, 64, 512


def _make_sc_gather():
    info = plsc.get_sparse_core_info()
    NC, NS = info.num_cores, info.num_subcores
    NW = NC * NS
    b_per_w = B // NW
    mesh = plsc.VectorSubcoreMesh(core_axis_name="c", subcore_axis_name="s")

    @functools.partial(
        pl.kernel,
        mesh=mesh,
        out_type=jax.ShapeDtypeStruct((B, D), jnp.float32),
        scratch_types=[
            pltpu.VMEM((b_per_w,), jnp.int32),
            pltpu.VMEM((b_per_w, D), jnp.float32),
            pltpu.SemaphoreType.DMA,
        ],
        compiler_params=pltpu.CompilerParams(use_tc_tiling_on_sc=False),
    )
    def gather_kernel(idx_hbm, table_hbm, out_hbm, idx_v, rows_v, sem):
        wid = lax.axis_index("s") * NC + lax.axis_index("c")
        base = wid * b_per_w
        pltpu.sync_copy(idx_hbm.at[pl.ds(base, b_per_w)], idx_v)
        pltpu.async_copy(table_hbm.at[idx_v], rows_v, sem).wait()
        pltpu.sync_copy(rows_v, out_hbm.at[pl.ds(base, b_per_w)])

    return gather_kernel


_sc_gather = _make_sc_gather()


BK = 1024


def _mlp_body(e_ref, W1_ref, b1_ref, W2_ref, b2_ref, W3_ref, b3_ref, out_ref):
    e = e_ref[...]
    h = jnp.tanh(
        jax.lax.dot_general(e, W1_ref[...], (((1,), (0,)), ((), ())),
                            preferred_element_type=jnp.float32)
        + b1_ref[...])
    h = jnp.tanh(
        jax.lax.dot_general(h, W2_ref[...], (((1,), (0,)), ((), ())),
                            preferred_element_type=jnp.float32)
        + b2_ref[...])
    out_ref[...] = (
        jax.lax.dot_general(h, W3_ref[...], (((1,), (0,)), ((), ())),
                            preferred_element_type=jnp.float32)
        + b3_ref[...])


def _mlp(e, W1, b1, W2, b2, W3, b3):
    grid = (B // BK,)
    return pl.pallas_call(
        _mlp_body,
        grid=grid,
        in_specs=[
            pl.BlockSpec((BK, D), lambda i: (i, 0)),
            pl.BlockSpec((D, H), lambda i: (0, 0)),
            pl.BlockSpec((1, H), lambda i: (0, 0)),
            pl.BlockSpec((H, H), lambda i: (0, 0)),
            pl.BlockSpec((1, H), lambda i: (0, 0)),
            pl.BlockSpec((H, 1), lambda i: (0, 0)),
            pl.BlockSpec((1, 1), lambda i: (0, 0)),
        ],
        out_specs=pl.BlockSpec((BK, 1), lambda i: (i, 0)),
        out_shape=jax.ShapeDtypeStruct((B, 1), jnp.float32),
    )(e, W1, b1, W2, b2, W3, b3)


def kernel(x, table, W1, b1, W2, b2, W3, b3):
    idx = jnp.reshape(x, (B,)).astype(jnp.int32)
    e = _sc_gather(idx, table)
    return _mlp(e, W1, b1.reshape(1, H), W2, b2.reshape(1, H),
                W3, b3.reshape(1, 1))
